# Initial kernel scaffold; baseline (speedup 1.0000x reference)
import jax
import jax.numpy as jnp
from jax.experimental import pallas as pl


def kernel(x, router_W, Wfc, Wg, Wp, Wsfc, Wsg, Wsp):
    raise NotImplementedError("write your pallas kernel here")


# speedup vs baseline: 1.0341x; 1.0341x over previous
import functools

import jax
import jax.numpy as jnp
from jax import lax
from jax.experimental import pallas as pl
from jax.experimental.pallas import tpu as pltpu
from jax.experimental.pallas import tpu_sc as plsc

D = 1024
E = 8
H = 4096
S = 2048
BLK = 128
NB = 40
G = NB * BLK
HT = 512
NH = H // HT
NSB = 16

_f32 = jnp.float32
_i32 = jnp.int32


def _router_body(x_ref, w_ref, pos0_ref, pos1_ref, w1_ref, w2_ref,
                 cnt_ref, lb_ref, g_ref, m_ref, c_ref):
    x = x_ref[...]
    logits = jnp.dot(x, w_ref[...], preferred_element_type=_f32)
    mx = jnp.max(logits, axis=1, keepdims=True)
    ex = jnp.exp(logits - mx)
    scores = ex / jnp.sum(ex, axis=1, keepdims=True)

    e_ids = lax.broadcasted_iota(_i32, (S, E), 1)
    s1 = jnp.max(scores, axis=1, keepdims=True)
    i1 = jnp.min(jnp.where(scores == s1, e_ids, E), axis=1, keepdims=True)
    masked = jnp.where(e_ids == i1, -1.0, scores)
    s2 = jnp.max(masked, axis=1, keepdims=True)
    i2 = jnp.min(jnp.where(masked == s2, e_ids, E), axis=1, keepdims=True)

    m1 = (e_ids == i1).astype(_f32)
    m2 = (e_ids == i2).astype(_f32)
    m_ref[...] = m1 + m2

    CH = 256
    for i in range(S // CH):
        rows = lax.broadcasted_iota(_i32, (CH, S), 0) + i * CH
        cols = lax.broadcasted_iota(_i32, (CH, S), 1)
        tril = (cols < rows).astype(_f32)
        c_ref[pl.ds(i * CH, CH), :] = jnp.dot(
            tril, m_ref[...], preferred_element_type=_f32)
    c = c_ref[...]

    cnt = jnp.sum(m_ref[...], axis=0, keepdims=True)
    cnt_pad = jnp.ceil(cnt / BLK) * BLK
    up8 = (lax.broadcasted_iota(_i32, (E, E), 0)
           < lax.broadcasted_iota(_i32, (E, E), 1)).astype(_f32)
    offs = jnp.dot(cnt_pad, up8, preferred_element_type=_f32)

    offs_b = jnp.broadcast_to(offs, (S, E))
    pos0_ref[...] = jnp.sum(m1 * (offs_b + c), axis=1, keepdims=True).astype(_i32)
    pos1_ref[...] = jnp.sum(m2 * (offs_b + c), axis=1, keepdims=True).astype(_i32)
    w1_ref[...] = s1
    w2_ref[...] = s2

    cnt_ref[...] = cnt.astype(_i32)
    p_e = jnp.sum(scores, axis=0, keepdims=True) / float(S)
    f_e = cnt / float(S * 2)
    lb_ref[...] = (E * jnp.sum(f_e * p_e, axis=1, keepdims=True) - 1.0)

    bid = (lax.broadcasted_iota(_i32, (64, E), 0) * BLK).astype(_f32)
    ind = (jnp.broadcast_to(offs, (64, E)) <= bid).astype(_i32)
    g_ref[...] = jnp.sum(ind, axis=1, keepdims=True) - 1


def _run_router(xf, router_W):
    outs = pl.pallas_call(
        _router_body,
        out_shape=(
            jax.ShapeDtypeStruct((S, 1), _i32),
            jax.ShapeDtypeStruct((S, 1), _i32),
            jax.ShapeDtypeStruct((S, 1), _f32),
            jax.ShapeDtypeStruct((S, 1), _f32),
            jax.ShapeDtypeStruct((1, E), _i32),
            jax.ShapeDtypeStruct((1, 1), _f32),
            jax.ShapeDtypeStruct((64, 1), _i32),
        ),
        scratch_shapes=[
            pltpu.VMEM((S, E), _f32),
            pltpu.VMEM((S, E), _f32),
        ],
    )(xf, router_W)
    return outs


_NTILES = 32
_TPC = S // 16
_SLOTS_PT = G // _NTILES
_INIT_PT = G // 16
_GCH = 80


def _dispatch_body(x_hbm, p0_hbm, p1_hbm, w1_hbm, w2_hbm, gx_hbm, rw_hbm,
                   srcsh, rwsh, zi, zf, posv, valv, tokv,
                   rwv, idxb, rowb, sem):
    c = lax.axis_index("c")
    s = lax.axis_index("s")

    for j in range(_INIT_PT // 16):
        zi[pl.ds(16 * j, 16)] = jnp.zeros((16,), _i32)
        zf[pl.ds(16 * j, 16)] = jnp.zeros((16,), _f32)
    pltpu.sync_copy(zi, srcsh.at[pl.ds(s * _INIT_PT, _INIT_PT)])
    pltpu.sync_copy(zf, rwsh.at[pl.ds(s * _INIT_PT, _INIT_PT)])
    plsc.subcore_barrier()

    tb = s * _TPC
    for j in range(_TPC // 16):
        tokv[pl.ds(16 * j, 16)] = lax.iota(_i32, 16) + (tb + 16 * j)
    for (p_hbm, w_hbm) in ((p0_hbm, w1_hbm), (p1_hbm, w2_hbm)):
        pltpu.sync_copy(p_hbm.at[pl.ds(tb, _TPC)], posv)
        pltpu.sync_copy(w_hbm.at[pl.ds(tb, _TPC)], valv)
        pltpu.sync_copy(tokv, srcsh.at[posv], add=True)
        pltpu.sync_copy(valv, rwsh.at[posv], add=True)
    plsc.subcore_barrier()

    base = c * (G // 2) + s * _SLOTS_PT
    pltpu.sync_copy(rwsh.at[pl.ds(base, _SLOTS_PT)], rwv)
    pltpu.sync_copy(rwv, rw_hbm.at[pl.ds(base, _SLOTS_PT)])
    for k in range(_SLOTS_PT // _GCH):
        pltpu.sync_copy(srcsh.at[pl.ds(base + k * _GCH, _GCH)], idxb)
        pltpu.async_copy(x_hbm.at[idxb], rowb, sem).wait()
        pltpu.sync_copy(rowb, gx_hbm.at[pl.ds(base + k * _GCH, _GCH)])


def _run_dispatch(xf, pos0, pos1, w1, w2):
    mesh = plsc.VectorSubcoreMesh(core_axis_name="c", subcore_axis_name="s")
    k = functools.partial(
        pl.kernel,
        mesh=mesh,
        out_type=(
            jax.ShapeDtypeStruct((G, D), _f32),
            jax.ShapeDtypeStruct((G,), _f32),
        ),
        scratch_types=[
            pltpu.VMEM_SHARED((G,), _i32),
            pltpu.VMEM_SHARED((G,), _f32),
            pltpu.VMEM((_INIT_PT,), _i32),
            pltpu.VMEM((_INIT_PT,), _f32),
            pltpu.VMEM((_TPC,), _i32),
            pltpu.VMEM((_TPC,), _f32),
            pltpu.VMEM((_TPC,), _i32),
            pltpu.VMEM((_SLOTS_PT,), _f32),
            pltpu.VMEM((_GCH,), _i32),
            pltpu.VMEM((_GCH, D), _f32),
            pltpu.SemaphoreType.DMA,
        ],
    )(_dispatch_body)
    return k(xf, pos0, pos1, w1, w2)


def _k1_body(g_ref, gx_ref, wfc_ref, wg_ref, h_ref):
    xb = gx_ref[...]
    a = jnp.dot(xb, wfc_ref[0], preferred_element_type=_f32)
    z = jnp.dot(xb, wg_ref[0], preferred_element_type=_f32)
    h_ref[...] = a * z * (1.0 / (1.0 + jnp.exp(-z)))


def _run_k1(g, gx, Wfc, Wg):
    return pl.pallas_call(
        _k1_body,
        grid_spec=pltpu.PrefetchScalarGridSpec(
            num_scalar_prefetch=1,
            grid=(NH, NB),
            in_specs=[
                pl.BlockSpec((BLK, D), lambda h, b, g: (b, 0)),
                pl.BlockSpec((1, D, HT), lambda h, b, g: (g[b], 0, h)),
                pl.BlockSpec((1, D, HT), lambda h, b, g: (g[b], 0, h)),
            ],
            out_specs=pl.BlockSpec((BLK, HT), lambda h, b, g: (b, h)),
        ),
        out_shape=jax.ShapeDtypeStruct((G, H), _f32),
    )(g, gx, Wfc, Wg)


def _k2_body(g_ref, h_ref, wp_ref, rw_ref, r_ref):
    r = jnp.dot(h_ref[...], wp_ref[0], preferred_element_type=_f32)
    r_ref[...] = r * rw_ref[:, 0:1]


def _run_k2(g, Hm, Wp, rw2d):
    return pl.pallas_call(
        _k2_body,
        grid_spec=pltpu.PrefetchScalarGridSpec(
            num_scalar_prefetch=1,
            grid=(NB,),
            in_specs=[
                pl.BlockSpec((BLK, H), lambda b, g: (b, 0)),
                pl.BlockSpec((1, H, D), lambda b, g: (g[b], 0, 0)),
                pl.BlockSpec((BLK, 128), lambda b, g: (b, 0)),
            ],
            out_specs=pl.BlockSpec((BLK, D), lambda b, g: (b, 0)),
        ),
        out_shape=jax.ShapeDtypeStruct((G, D), _f32),
    )(g, Hm, Wp, rw2d)


def _ks1_body(x_ref, wfc_ref, wg_ref, h_ref):
    xb = x_ref[...]
    a = jnp.dot(xb, wfc_ref[...], preferred_element_type=_f32)
    z = jnp.dot(xb, wg_ref[...], preferred_element_type=_f32)
    h_ref[...] = a * z * (1.0 / (1.0 + jnp.exp(-z)))


def _run_ks1(xf, Wsfc, Wsg):
    return pl.pallas_call(
        _ks1_body,
        grid=(NH, NSB),
        in_specs=[
            pl.BlockSpec((BLK, D), lambda h, b: (b, 0)),
            pl.BlockSpec((D, HT), lambda h, b: (0, h)),
            pl.BlockSpec((D, HT), lambda h, b: (0, h)),
        ],
        out_specs=pl.BlockSpec((BLK, HT), lambda h, b: (b, h)),
        out_shape=jax.ShapeDtypeStruct((S, H), _f32),
    )(xf, Wsfc, Wsg)


def _ks2_body(h_ref, wp_ref, r_ref):
    r_ref[...] = jnp.dot(h_ref[...], wp_ref[...], preferred_element_type=_f32)


def _run_ks2(Hs, Wsp):
    return pl.pallas_call(
        _ks2_body,
        grid=(NSB,),
        in_specs=[
            pl.BlockSpec((BLK, H), lambda b: (b, 0)),
            pl.BlockSpec((H, D), lambda b: (0, 0)),
        ],
        out_specs=pl.BlockSpec((BLK, D), lambda b: (b, 0)),
        out_shape=jax.ShapeDtypeStruct((S, D), _f32),
    )(Hs, Wsp)


_TOK_PT = S // _NTILES
_CCH = 16


def _combine_body(r_hbm, rs_hbm, p0_hbm, p1_hbm, y_hbm,
                  idx0, idx1, b0, b1, bs, ob, sem):
    c = lax.axis_index("c")
    s = lax.axis_index("s")
    base = (s * 2 + c) * _TOK_PT
    for k in range(_TOK_PT // _CCH):
        tb = base + k * _CCH
        pltpu.sync_copy(p0_hbm.at[pl.ds(tb, _CCH)], idx0)
        pltpu.sync_copy(p1_hbm.at[pl.ds(tb, _CCH)], idx1)
        pltpu.async_copy(r_hbm.at[idx0], b0, sem).wait()
        pltpu.async_copy(r_hbm.at[idx1], b1, sem).wait()
        pltpu.sync_copy(rs_hbm.at[pl.ds(tb, _CCH)], bs)
        for i in range(_CCH):
            def _add(j, _, i=i):
                sl = pl.ds(j * 16, 16)
                ob[i, sl] = b0[i, sl] + b1[i, sl] + bs[i, sl]
                return 0
            lax.fori_loop(0, D // 16, _add, 0)
        pltpu.sync_copy(ob, y_hbm.at[pl.ds(tb, _CCH)])


def _run_combine(R, Rs, pos0, pos1):
    mesh = plsc.VectorSubcoreMesh(core_axis_name="c", subcore_axis_name="s")
    k = functools.partial(
        pl.kernel,
        mesh=mesh,
        out_type=jax.ShapeDtypeStruct((S, D), _f32),
        scratch_types=[
            pltpu.VMEM((_CCH,), _i32),
            pltpu.VMEM((_CCH,), _i32),
            pltpu.VMEM((_CCH, D), _f32),
            pltpu.VMEM((_CCH, D), _f32),
            pltpu.VMEM((_CCH, D), _f32),
            pltpu.VMEM((_CCH, D), _f32),
            pltpu.SemaphoreType.DMA,
        ],
    )(_combine_body)
    return k(R, Rs, pos0, pos1)


def kernel(x, router_W, Wfc, Wg, Wp, Wsfc, Wsg, Wsp):
    B, S_, D_ = x.shape
    xf = x.reshape(S_, D_)

    (pos0c, pos1c, w1c, w2c, cnt2, lb2, g2) = _run_router(xf, router_W)
    pos0 = pos0c.reshape(S)
    pos1 = pos1c.reshape(S)
    g = g2.reshape(64)[:NB]

    gx, rw = _run_dispatch(xf, pos0, pos1, w1c.reshape(S), w2c.reshape(S))
    rw2d = jnp.broadcast_to(rw[:, None], (G, 128))

    Hm = _run_k1(g, gx, Wfc, Wg)
    R = _run_k2(g, Hm, Wp, rw2d)

    Hs = _run_ks1(xf, Wsfc, Wsg)
    Rs = _run_ks2(Hs, Wsp)

    y2 = _run_combine(R, Rs, pos0, pos1)

    y = y2.reshape(B, S_, D_)
    lb_loss = lb2.reshape(())
    counts = cnt2.reshape(E)
    return (y, lb_loss, counts)


# speedup vs baseline: 1.3927x; 1.3469x over previous
import functools

import jax
import jax.numpy as jnp
from jax import lax
from jax.experimental import pallas as pl
from jax.experimental.pallas import tpu as pltpu
from jax.experimental.pallas import tpu_sc as plsc

D = 1024
E = 8
H = 4096
S = 2048
BLK = 256
NB = 23
G = NB * BLK
HT = 1024
NH = H // HT
NSB = S // BLK

_f32 = jnp.float32
_i32 = jnp.int32


def _router_body(x_ref, w_ref, pos0_ref, pos1_ref, w1_ref, w2_ref,
                 cnt_ref, lb_ref, g_ref, m_ref, c_ref):
    x = x_ref[...]
    logits = jnp.dot(x, w_ref[...], preferred_element_type=_f32)
    mx = jnp.max(logits, axis=1, keepdims=True)
    ex = jnp.exp(logits - mx)
    scores = ex / jnp.sum(ex, axis=1, keepdims=True)

    e_ids = lax.broadcasted_iota(_i32, (S, E), 1)
    s1 = jnp.max(scores, axis=1, keepdims=True)
    i1 = jnp.min(jnp.where(scores == s1, e_ids, E), axis=1, keepdims=True)
    masked = jnp.where(e_ids == i1, -1.0, scores)
    s2 = jnp.max(masked, axis=1, keepdims=True)
    i2 = jnp.min(jnp.where(masked == s2, e_ids, E), axis=1, keepdims=True)

    m1 = (e_ids == i1).astype(_f32)
    m2 = (e_ids == i2).astype(_f32)
    m_ref[...] = m1 + m2

    CH = 256
    for i in range(S // CH):
        rows = lax.broadcasted_iota(_i32, (CH, S), 0) + i * CH
        cols = lax.broadcasted_iota(_i32, (CH, S), 1)
        tril = (cols < rows).astype(_f32)
        c_ref[pl.ds(i * CH, CH), :] = jnp.dot(
            tril, m_ref[...], preferred_element_type=_f32)
    c = c_ref[...]

    cnt = jnp.sum(m_ref[...], axis=0, keepdims=True)
    cnt_pad = jnp.ceil(cnt / BLK) * BLK
    up8 = (lax.broadcasted_iota(_i32, (E, E), 0)
           < lax.broadcasted_iota(_i32, (E, E), 1)).astype(_f32)
    offs = jnp.dot(cnt_pad, up8, preferred_element_type=_f32)

    offs_b = jnp.broadcast_to(offs, (S, E))
    pos0_ref[...] = jnp.sum(m1 * (offs_b + c), axis=1, keepdims=True).astype(_i32)
    pos1_ref[...] = jnp.sum(m2 * (offs_b + c), axis=1, keepdims=True).astype(_i32)
    w1_ref[...] = s1
    w2_ref[...] = s2

    cnt_ref[...] = cnt.astype(_i32)
    p_e = jnp.sum(scores, axis=0, keepdims=True) / float(S)
    f_e = cnt / float(S * 2)
    lb_ref[...] = (E * jnp.sum(f_e * p_e, axis=1, keepdims=True) - 1.0)

    bid = (lax.broadcasted_iota(_i32, (64, E), 0) * BLK).astype(_f32)
    ind = (jnp.broadcast_to(offs, (64, E)) <= bid).astype(_i32)
    g_ref[...] = jnp.sum(ind, axis=1, keepdims=True) - 1


def _run_router(xf, router_W):
    outs = pl.pallas_call(
        _router_body,
        out_shape=(
            jax.ShapeDtypeStruct((S, 1), _i32),
            jax.ShapeDtypeStruct((S, 1), _i32),
            jax.ShapeDtypeStruct((S, 1), _f32),
            jax.ShapeDtypeStruct((S, 1), _f32),
            jax.ShapeDtypeStruct((1, E), _i32),
            jax.ShapeDtypeStruct((1, 1), _f32),
            jax.ShapeDtypeStruct((64, 1), _i32),
        ),
        scratch_shapes=[
            pltpu.VMEM((S, E), _f32),
            pltpu.VMEM((S, E), _f32),
        ],
    )(xf, router_W)
    return outs


_NTILES = 32
_TPC = S // 16
_SLOTS_PT = G // _NTILES
_GCH = 48
_GOFF = (0, 48, 96, 136)


def _clamp16(ref, n):
    for j in range(n // 16):
        sl = pl.ds(16 * j, 16)
        v = ref[sl]
        ref[sl] = jnp.minimum(jnp.maximum(v, 0), S - 1)


def _dispatch_body(x_hbm, p0_hbm, p1_hbm, w1_hbm, w2_hbm, gx_hbm, rw_hbm,
                   srcsh, rwsh, posv, valv, tokv, rwv,
                   ia, ib, ra, rb, sem):
    c = lax.axis_index("c")
    s = lax.axis_index("s")

    tb = s * _TPC
    for j in range(_TPC // 16):
        tokv[pl.ds(16 * j, 16)] = lax.iota(_i32, 16) + (tb + 16 * j)
    for (p_hbm, w_hbm) in ((p0_hbm, w1_hbm), (p1_hbm, w2_hbm)):
        pltpu.sync_copy(p_hbm.at[pl.ds(tb, _TPC)], posv)
        pltpu.sync_copy(w_hbm.at[pl.ds(tb, _TPC)], valv)
        pltpu.sync_copy(tokv, srcsh.at[posv])
        pltpu.sync_copy(valv, rwsh.at[posv])
    plsc.subcore_barrier()

    base = c * (G // 2) + s * _SLOTS_PT
    pltpu.sync_copy(rwsh.at[pl.ds(base, _SLOTS_PT)], rwv)
    pltpu.sync_copy(rwv, rw_hbm.at[pl.ds(base, _SLOTS_PT)])
    ibufs = (ia, ib)
    rbufs = (ra, rb)
    pltpu.sync_copy(srcsh.at[pl.ds(base + _GOFF[0], _GCH)], ia)
    _clamp16(ia, _GCH)
    cp = pltpu.async_copy(x_hbm.at[ia], ra, sem)
    for k in range(1, len(_GOFF)):
        nib = ibufs[k % 2]
        nrb = rbufs[k % 2]
        pltpu.sync_copy(srcsh.at[pl.ds(base + _GOFF[k], _GCH)], nib)
        _clamp16(nib, _GCH)
        cp.wait()
        cp = pltpu.async_copy(x_hbm.at[nib], nrb, sem)
        pltpu.sync_copy(rbufs[(k - 1) % 2],
                        gx_hbm.at[pl.ds(base + _GOFF[k - 1], _GCH)])
    cp.wait()
    pltpu.sync_copy(rbufs[(len(_GOFF) - 1) % 2],
                    gx_hbm.at[pl.ds(base + _GOFF[-1], _GCH)])


def _run_dispatch(xf, pos0, pos1, w1, w2):
    mesh = plsc.VectorSubcoreMesh(core_axis_name="c", subcore_axis_name="s")
    k = functools.partial(
        pl.kernel,
        mesh=mesh,
        out_type=(
            jax.ShapeDtypeStruct((G, D), _f32),
            jax.ShapeDtypeStruct((G,), _f32),
        ),
        scratch_types=[
            pltpu.VMEM_SHARED((G,), _i32),
            pltpu.VMEM_SHARED((G,), _f32),
            pltpu.VMEM((_TPC,), _i32),
            pltpu.VMEM((_TPC,), _f32),
            pltpu.VMEM((_TPC,), _i32),
            pltpu.VMEM((_SLOTS_PT,), _f32),
            pltpu.VMEM((_GCH,), _i32),
            pltpu.VMEM((_GCH,), _i32),
            pltpu.VMEM((_GCH, D), _f32),
            pltpu.VMEM((_GCH, D), _f32),
            pltpu.SemaphoreType.DMA,
        ],
    )(_dispatch_body)
    return k(xf, pos0, pos1, w1, w2)


def _k1_body(g_ref, gx_ref, wfc_ref, wg_ref, h_ref):
    xb = gx_ref[...]
    a = jnp.dot(xb, wfc_ref[0], preferred_element_type=_f32)
    z = jnp.dot(xb, wg_ref[0], preferred_element_type=_f32)
    h_ref[...] = a * z * (1.0 / (1.0 + jnp.exp(-z)))


def _run_k1(g, gx, Wfc, Wg):
    return pl.pallas_call(
        _k1_body,
        grid_spec=pltpu.PrefetchScalarGridSpec(
            num_scalar_prefetch=1,
            grid=(NH, NB),
            in_specs=[
                pl.BlockSpec((BLK, D), lambda h, b, g: (b, 0)),
                pl.BlockSpec((1, D, HT), lambda h, b, g: (g[b], 0, h)),
                pl.BlockSpec((1, D, HT), lambda h, b, g: (g[b], 0, h)),
            ],
            out_specs=pl.BlockSpec((BLK, HT), lambda h, b, g: (b, h)),
        ),
        out_shape=jax.ShapeDtypeStruct((G, H), _f32),
    )(g, gx, Wfc, Wg)


def _k2_body(g_ref, h_ref, wp_ref, rw_ref, r_ref):
    r = jnp.dot(h_ref[...], wp_ref[0], preferred_element_type=_f32)
    r_ref[...] = r * rw_ref[:, 0:1]


def _run_k2(g, Hm, Wp, rw2d):
    return pl.pallas_call(
        _k2_body,
        grid_spec=pltpu.PrefetchScalarGridSpec(
            num_scalar_prefetch=1,
            grid=(NB,),
            in_specs=[
                pl.BlockSpec((BLK, H), lambda b, g: (b, 0)),
                pl.BlockSpec((1, H, D), lambda b, g: (g[b], 0, 0)),
                pl.BlockSpec((BLK, 128), lambda b, g: (b, 0)),
            ],
            out_specs=pl.BlockSpec((BLK, D), lambda b, g: (b, 0)),
        ),
        out_shape=jax.ShapeDtypeStruct((G, D), _f32),
    )(g, Hm, Wp, rw2d)


def _ks1_body(x_ref, wfc_ref, wg_ref, h_ref):
    xb = x_ref[...]
    a = jnp.dot(xb, wfc_ref[...], preferred_element_type=_f32)
    z = jnp.dot(xb, wg_ref[...], preferred_element_type=_f32)
    h_ref[...] = a * z * (1.0 / (1.0 + jnp.exp(-z)))


def _run_ks1(xf, Wsfc, Wsg):
    return pl.pallas_call(
        _ks1_body,
        grid=(NH, NSB),
        in_specs=[
            pl.BlockSpec((BLK, D), lambda h, b: (b, 0)),
            pl.BlockSpec((D, HT), lambda h, b: (0, h)),
            pl.BlockSpec((D, HT), lambda h, b: (0, h)),
        ],
        out_specs=pl.BlockSpec((BLK, HT), lambda h, b: (b, h)),
        out_shape=jax.ShapeDtypeStruct((S, H), _f32),
    )(xf, Wsfc, Wsg)


def _ks2_body(h_ref, wp_ref, r_ref):
    r_ref[...] = jnp.dot(h_ref[...], wp_ref[...], preferred_element_type=_f32)


def _run_ks2(Hs, Wsp):
    return pl.pallas_call(
        _ks2_body,
        grid=(NSB,),
        in_specs=[
            pl.BlockSpec((BLK, H), lambda b: (b, 0)),
            pl.BlockSpec((H, D), lambda b: (0, 0)),
        ],
        out_specs=pl.BlockSpec((BLK, D), lambda b: (b, 0)),
        out_shape=jax.ShapeDtypeStruct((S, D), _f32),
    )(Hs, Wsp)


_TOK_PT = S // _NTILES
_CCH = 16


def _combine_body(r_hbm, rs_hbm, p0_hbm, p1_hbm, y_hbm,
                  idx0, idx1, b0, b1, bs, sem):
    c = lax.axis_index("c")
    s = lax.axis_index("s")
    base = (s * 2 + c) * _TOK_PT
    for k in range(_TOK_PT // _CCH):
        tb = base + k * _CCH
        pltpu.sync_copy(p0_hbm.at[pl.ds(tb, _CCH)], idx0)
        pltpu.sync_copy(p1_hbm.at[pl.ds(tb, _CCH)], idx1)
        cp0 = pltpu.async_copy(r_hbm.at[idx0], b0, sem)
        cp1 = pltpu.async_copy(r_hbm.at[idx1], b1, sem)
        pltpu.sync_copy(rs_hbm.at[pl.ds(tb, _CCH)], bs)
        cp0.wait()
        cp1.wait()
        for i in range(_CCH):
            def _add(j, _, i=i):
                for u in range(4):
                    sl = pl.ds((j * 4 + u) * 16, 16)
                    b0[i, sl] = b0[i, sl] + b1[i, sl] + bs[i, sl]
                return 0
            lax.fori_loop(0, D // 64, _add, 0)
        pltpu.sync_copy(b0, y_hbm.at[pl.ds(tb, _CCH)])


def _run_combine(R, Rs, pos0, pos1):
    mesh = plsc.VectorSubcoreMesh(core_axis_name="c", subcore_axis_name="s")
    k = functools.partial(
        pl.kernel,
        mesh=mesh,
        out_type=jax.ShapeDtypeStruct((S, D), _f32),
        scratch_types=[
            pltpu.VMEM((_CCH,), _i32),
            pltpu.VMEM((_CCH,), _i32),
            pltpu.VMEM((_CCH, D), _f32),
            pltpu.VMEM((_CCH, D), _f32),
            pltpu.VMEM((_CCH, D), _f32),
            pltpu.SemaphoreType.DMA,
        ],
    )(_combine_body)
    return k(R, Rs, pos0, pos1)


def kernel(x, router_W, Wfc, Wg, Wp, Wsfc, Wsg, Wsp):
    B, S_, D_ = x.shape
    xf = x.reshape(S_, D_)

    (pos0c, pos1c, w1c, w2c, cnt2, lb2, g2) = _run_router(xf, router_W)
    pos0 = pos0c.reshape(S)
    pos1 = pos1c.reshape(S)
    g = g2.reshape(64)[:NB]

    gx, rw = _run_dispatch(xf, pos0, pos1, w1c.reshape(S), w2c.reshape(S))
    rw2d = jnp.broadcast_to(rw[:, None], (G, 128))

    Hm = _run_k1(g, gx, Wfc, Wg)
    R = _run_k2(g, Hm, Wp, rw2d)

    Hs = _run_ks1(xf, Wsfc, Wsg)
    Rs = _run_ks2(Hs, Wsp)

    y2 = _run_combine(R, Rs, pos0, pos1)

    y = y2.reshape(B, S_, D_)
    lb_loss = lb2.reshape(())
    counts = cnt2.reshape(E)
    return (y, lb_loss, counts)


# speedup vs baseline: 1.4152x; 1.0161x over previous
import functools

import jax
import jax.numpy as jnp
from jax import lax
from jax.experimental import pallas as pl
from jax.experimental.pallas import tpu as pltpu
from jax.experimental.pallas import tpu_sc as plsc

D = 1024
E = 8
H = 4096
S = 2048
BLK = 256
NB = 23
G = NB * BLK
HT = 1024
NH = H // HT
NSB = S // BLK

_f32 = jnp.float32
_i32 = jnp.int32


def _router_body(x_ref, w_ref, pos0_ref, pos1_ref, w1_ref, w2_ref,
                 cnt_ref, lb_ref, g_ref, m_ref, c_ref):
    x = x_ref[...]
    logits = jnp.dot(x, w_ref[...], preferred_element_type=_f32)
    mx = jnp.max(logits, axis=1, keepdims=True)
    ex = jnp.exp(logits - mx)
    scores = ex / jnp.sum(ex, axis=1, keepdims=True)

    e_ids = lax.broadcasted_iota(_i32, (S, E), 1)
    s1 = jnp.max(scores, axis=1, keepdims=True)
    i1 = jnp.min(jnp.where(scores == s1, e_ids, E), axis=1, keepdims=True)
    masked = jnp.where(e_ids == i1, -1.0, scores)
    s2 = jnp.max(masked, axis=1, keepdims=True)
    i2 = jnp.min(jnp.where(masked == s2, e_ids, E), axis=1, keepdims=True)

    m1 = (e_ids == i1).astype(_f32)
    m2 = (e_ids == i2).astype(_f32)
    m_ref[...] = m1 + m2

    CH = 256
    for i in range(S // CH):
        rows = lax.broadcasted_iota(_i32, (CH, S), 0) + i * CH
        cols = lax.broadcasted_iota(_i32, (CH, S), 1)
        tril = (cols < rows).astype(_f32)
        c_ref[pl.ds(i * CH, CH), :] = jnp.dot(
            tril, m_ref[...], preferred_element_type=_f32)
    c = c_ref[...]

    cnt = jnp.sum(m_ref[...], axis=0, keepdims=True)
    cnt_pad = jnp.ceil(cnt / BLK) * BLK
    up8 = (lax.broadcasted_iota(_i32, (E, E), 0)
           < lax.broadcasted_iota(_i32, (E, E), 1)).astype(_f32)
    offs = jnp.dot(cnt_pad, up8, preferred_element_type=_f32)

    offs_b = jnp.broadcast_to(offs, (S, E))
    pos0_ref[...] = jnp.sum(m1 * (offs_b + c), axis=1, keepdims=True).astype(_i32)
    pos1_ref[...] = jnp.sum(m2 * (offs_b + c), axis=1, keepdims=True).astype(_i32)
    w1_ref[...] = s1
    w2_ref[...] = s2

    cnt_ref[...] = cnt.astype(_i32)
    p_e = jnp.sum(scores, axis=0, keepdims=True) / float(S)
    f_e = cnt / float(S * 2)
    lb_ref[...] = (E * jnp.sum(f_e * p_e, axis=1, keepdims=True) - 1.0)

    bid = (lax.broadcasted_iota(_i32, (64, E), 0) * BLK).astype(_f32)
    ind = (jnp.broadcast_to(offs, (64, E)) <= bid).astype(_i32)
    g_ref[...] = jnp.sum(ind, axis=1, keepdims=True) - 1


def _run_router(xf, router_W):
    outs = pl.pallas_call(
        _router_body,
        out_shape=(
            jax.ShapeDtypeStruct((S, 1), _i32),
            jax.ShapeDtypeStruct((S, 1), _i32),
            jax.ShapeDtypeStruct((S, 1), _f32),
            jax.ShapeDtypeStruct((S, 1), _f32),
            jax.ShapeDtypeStruct((1, E), _i32),
            jax.ShapeDtypeStruct((1, 1), _f32),
            jax.ShapeDtypeStruct((64, 1), _i32),
        ),
        scratch_shapes=[
            pltpu.VMEM((S, E), _f32),
            pltpu.VMEM((S, E), _f32),
        ],
    )(xf, router_W)
    return outs


_NTILES = 32
_TPC = S // 16
_SLOTS_PT = G // _NTILES
_GCH = 48
_GOFF = (0, 48, 96, 136)


def _clamp16(ref, n):
    for j in range(n // 16):
        sl = pl.ds(16 * j, 16)
        v = ref[sl]
        ref[sl] = jnp.minimum(jnp.maximum(v, 0), S - 1)


def _dispatch_body(x_hbm, p0_hbm, p1_hbm, w1_hbm, w2_hbm, gx_hbm, rw_hbm,
                   srcsh, rwsh, posv, valv, tokv, rwv,
                   ia, ib, ra, rb, sem):
    c = lax.axis_index("c")
    s = lax.axis_index("s")

    tb = s * _TPC
    for j in range(_TPC // 16):
        tokv[pl.ds(16 * j, 16)] = lax.iota(_i32, 16) + (tb + 16 * j)
    for (p_hbm, w_hbm) in ((p0_hbm, w1_hbm), (p1_hbm, w2_hbm)):
        pltpu.sync_copy(p_hbm.at[pl.ds(tb, _TPC)], posv)
        pltpu.sync_copy(w_hbm.at[pl.ds(tb, _TPC)], valv)
        pltpu.sync_copy(tokv, srcsh.at[posv])
        pltpu.sync_copy(valv, rwsh.at[posv])
    plsc.subcore_barrier()

    base = c * (G // 2) + s * _SLOTS_PT
    pltpu.sync_copy(rwsh.at[pl.ds(base, _SLOTS_PT)], rwv)
    pltpu.sync_copy(rwv, rw_hbm.at[pl.ds(base, _SLOTS_PT)])
    ibufs = (ia, ib)
    rbufs = (ra, rb)
    pltpu.sync_copy(srcsh.at[pl.ds(base + _GOFF[0], _GCH)], ia)
    _clamp16(ia, _GCH)
    cp = pltpu.async_copy(x_hbm.at[ia], ra, sem)
    for k in range(1, len(_GOFF)):
        nib = ibufs[k % 2]
        nrb = rbufs[k % 2]
        pltpu.sync_copy(srcsh.at[pl.ds(base + _GOFF[k], _GCH)], nib)
        _clamp16(nib, _GCH)
        cp.wait()
        cp = pltpu.async_copy(x_hbm.at[nib], nrb, sem)
        pltpu.sync_copy(rbufs[(k - 1) % 2],
                        gx_hbm.at[pl.ds(base + _GOFF[k - 1], _GCH)])
    cp.wait()
    pltpu.sync_copy(rbufs[(len(_GOFF) - 1) % 2],
                    gx_hbm.at[pl.ds(base + _GOFF[-1], _GCH)])


def _run_dispatch(xf, pos0, pos1, w1, w2):
    mesh = plsc.VectorSubcoreMesh(core_axis_name="c", subcore_axis_name="s")
    k = functools.partial(
        pl.kernel,
        mesh=mesh,
        out_type=(
            jax.ShapeDtypeStruct((G, D), _f32),
            jax.ShapeDtypeStruct((G,), _f32),
        ),
        scratch_types=[
            pltpu.VMEM_SHARED((G,), _i32),
            pltpu.VMEM_SHARED((G,), _f32),
            pltpu.VMEM((_TPC,), _i32),
            pltpu.VMEM((_TPC,), _f32),
            pltpu.VMEM((_TPC,), _i32),
            pltpu.VMEM((_SLOTS_PT,), _f32),
            pltpu.VMEM((_GCH,), _i32),
            pltpu.VMEM((_GCH,), _i32),
            pltpu.VMEM((_GCH, D), _f32),
            pltpu.VMEM((_GCH, D), _f32),
            pltpu.SemaphoreType.DMA,
        ],
    )(_dispatch_body)
    return k(xf, pos0, pos1, w1, w2)


_bf16 = jnp.bfloat16


def _k1_body(g_ref, gx_ref, wfc_ref, wg_ref, h_ref):
    xb = gx_ref[...].astype(_bf16)
    a = jnp.dot(xb, wfc_ref[0].astype(_bf16), preferred_element_type=_f32)
    z = jnp.dot(xb, wg_ref[0].astype(_bf16), preferred_element_type=_f32)
    h_ref[...] = (a * z * (1.0 / (1.0 + jnp.exp(-z)))).astype(_bf16)


def _run_k1(g, gx, Wfc, Wg):
    return pl.pallas_call(
        _k1_body,
        grid_spec=pltpu.PrefetchScalarGridSpec(
            num_scalar_prefetch=1,
            grid=(NH, NB),
            in_specs=[
                pl.BlockSpec((BLK, D), lambda h, b, g: (b, 0)),
                pl.BlockSpec((1, D, HT), lambda h, b, g: (g[b], 0, h)),
                pl.BlockSpec((1, D, HT), lambda h, b, g: (g[b], 0, h)),
            ],
            out_specs=pl.BlockSpec((BLK, HT), lambda h, b, g: (b, h)),
        ),
        out_shape=jax.ShapeDtypeStruct((G, H), _bf16),
    )(g, gx, Wfc, Wg)


def _k2_body(g_ref, h_ref, wp_ref, rw_ref, r_ref):
    r = jnp.dot(h_ref[...], wp_ref[0].astype(_bf16),
                preferred_element_type=_f32)
    r_ref[...] = r * rw_ref[:, 0:1]


def _run_k2(g, Hm, Wp, rw2d):
    return pl.pallas_call(
        _k2_body,
        grid_spec=pltpu.PrefetchScalarGridSpec(
            num_scalar_prefetch=1,
            grid=(NB,),
            in_specs=[
                pl.BlockSpec((BLK, H), lambda b, g: (b, 0)),
                pl.BlockSpec((1, H, D), lambda b, g: (g[b], 0, 0)),
                pl.BlockSpec((BLK, 128), lambda b, g: (b, 0)),
            ],
            out_specs=pl.BlockSpec((BLK, D), lambda b, g: (b, 0)),
        ),
        out_shape=jax.ShapeDtypeStruct((G, D), _f32),
    )(g, Hm, Wp, rw2d)


def _ks1_body(x_ref, wfc_ref, wg_ref, h_ref):
    xb = x_ref[...].astype(_bf16)
    a = jnp.dot(xb, wfc_ref[...].astype(_bf16), preferred_element_type=_f32)
    z = jnp.dot(xb, wg_ref[...].astype(_bf16), preferred_element_type=_f32)
    h_ref[...] = (a * z * (1.0 / (1.0 + jnp.exp(-z)))).astype(_bf16)


def _run_ks1(xf, Wsfc, Wsg):
    return pl.pallas_call(
        _ks1_body,
        grid=(NH, NSB),
        in_specs=[
            pl.BlockSpec((BLK, D), lambda h, b: (b, 0)),
            pl.BlockSpec((D, HT), lambda h, b: (0, h)),
            pl.BlockSpec((D, HT), lambda h, b: (0, h)),
        ],
        out_specs=pl.BlockSpec((BLK, HT), lambda h, b: (b, h)),
        out_shape=jax.ShapeDtypeStruct((S, H), _bf16),
    )(xf, Wsfc, Wsg)


def _ks2_body(h_ref, wp_ref, r_ref):
    r_ref[...] = jnp.dot(h_ref[...], wp_ref[...].astype(_bf16),
                         preferred_element_type=_f32)


def _run_ks2(Hs, Wsp):
    return pl.pallas_call(
        _ks2_body,
        grid=(NSB,),
        in_specs=[
            pl.BlockSpec((BLK, H), lambda b: (b, 0)),
            pl.BlockSpec((H, D), lambda b: (0, 0)),
        ],
        out_specs=pl.BlockSpec((BLK, D), lambda b: (b, 0)),
        out_shape=jax.ShapeDtypeStruct((S, D), _f32),
    )(Hs, Wsp)


_TOK_PT = S // _NTILES
_CCH = 16


def _combine_body(r_hbm, rs_hbm, p0_hbm, p1_hbm, y_hbm,
                  idx0, idx1, b0, b1, bs, sem):
    c = lax.axis_index("c")
    s = lax.axis_index("s")
    base = (s * 2 + c) * _TOK_PT
    for k in range(_TOK_PT // _CCH):
        tb = base + k * _CCH
        pltpu.sync_copy(p0_hbm.at[pl.ds(tb, _CCH)], idx0)
        pltpu.sync_copy(p1_hbm.at[pl.ds(tb, _CCH)], idx1)
        cp0 = pltpu.async_copy(r_hbm.at[idx0], b0, sem)
        cp1 = pltpu.async_copy(r_hbm.at[idx1], b1, sem)
        pltpu.sync_copy(rs_hbm.at[pl.ds(tb, _CCH)], bs)
        cp0.wait()
        cp1.wait()
        for i in range(_CCH):
            def _add(j, _, i=i):
                for u in range(4):
                    sl = pl.ds((j * 4 + u) * 16, 16)
                    b0[i, sl] = b0[i, sl] + b1[i, sl] + bs[i, sl]
                return 0
            lax.fori_loop(0, D // 64, _add, 0)
        pltpu.sync_copy(b0, y_hbm.at[pl.ds(tb, _CCH)])


def _run_combine(R, Rs, pos0, pos1):
    mesh = plsc.VectorSubcoreMesh(core_axis_name="c", subcore_axis_name="s")
    k = functools.partial(
        pl.kernel,
        mesh=mesh,
        out_type=jax.ShapeDtypeStruct((S, D), _f32),
        scratch_types=[
            pltpu.VMEM((_CCH,), _i32),
            pltpu.VMEM((_CCH,), _i32),
            pltpu.VMEM((_CCH, D), _f32),
            pltpu.VMEM((_CCH, D), _f32),
            pltpu.VMEM((_CCH, D), _f32),
            pltpu.SemaphoreType.DMA,
        ],
    )(_combine_body)
    return k(R, Rs, pos0, pos1)


def kernel(x, router_W, Wfc, Wg, Wp, Wsfc, Wsg, Wsp):
    B, S_, D_ = x.shape
    xf = x.reshape(S_, D_)

    (pos0c, pos1c, w1c, w2c, cnt2, lb2, g2) = _run_router(xf, router_W)
    pos0 = pos0c.reshape(S)
    pos1 = pos1c.reshape(S)
    g = g2.reshape(64)[:NB]

    gx, rw = _run_dispatch(xf, pos0, pos1, w1c.reshape(S), w2c.reshape(S))
    rw2d = jnp.broadcast_to(rw[:, None], (G, 128))

    Hs = _run_ks1(xf, Wsfc, Wsg)
    Rs = _run_ks2(Hs, Wsp)

    Hm = _run_k1(g, gx, Wfc, Wg)
    R = _run_k2(g, Hm, Wp, rw2d)

    y2 = _run_combine(R, Rs, pos0, pos1)

    y = y2.reshape(B, S_, D_)
    lb_loss = lb2.reshape(())
    counts = cnt2.reshape(E)
    return (y, lb_loss, counts)


# speedup vs baseline: 1.5289x; 1.0804x over previous
import functools

import jax
import jax.numpy as jnp
from jax import lax
from jax.experimental import pallas as pl
from jax.experimental.pallas import tpu as pltpu
from jax.experimental.pallas import tpu_sc as plsc

D = 1024
E = 8
H = 4096
S = 2048
BLK = 256
NB = 23
G = NB * BLK
HT = 2048
NH = H // HT
SBLK = 512
NSB = S // SBLK

_f32 = jnp.float32
_i32 = jnp.int32


def _router_body(x_ref, w_ref, pos0_ref, pos1_ref, w1_ref, w2_ref,
                 cnt_ref, lb_ref, g_ref, m_ref, c_ref):
    x = x_ref[...]
    logits = jnp.dot(x, w_ref[...], preferred_element_type=_f32)
    mx = jnp.max(logits, axis=1, keepdims=True)
    ex = jnp.exp(logits - mx)
    scores = ex / jnp.sum(ex, axis=1, keepdims=True)

    e_ids = lax.broadcasted_iota(_i32, (S, E), 1)
    s1 = jnp.max(scores, axis=1, keepdims=True)
    i1 = jnp.min(jnp.where(scores == s1, e_ids, E), axis=1, keepdims=True)
    masked = jnp.where(e_ids == i1, -1.0, scores)
    s2 = jnp.max(masked, axis=1, keepdims=True)
    i2 = jnp.min(jnp.where(masked == s2, e_ids, E), axis=1, keepdims=True)

    m1 = (e_ids == i1).astype(_f32)
    m2 = (e_ids == i2).astype(_f32)
    m_ref[...] = m1 + m2

    CH = 256
    for i in range(S // CH):
        rows = lax.broadcasted_iota(_i32, (CH, S), 0) + i * CH
        cols = lax.broadcasted_iota(_i32, (CH, S), 1)
        tril = (cols < rows).astype(_f32)
        c_ref[pl.ds(i * CH, CH), :] = jnp.dot(
            tril, m_ref[...], preferred_element_type=_f32)
    c = c_ref[...]

    cnt = jnp.sum(m_ref[...], axis=0, keepdims=True)
    cnt_pad = jnp.ceil(cnt / BLK) * BLK
    up8 = (lax.broadcasted_iota(_i32, (E, E), 0)
           < lax.broadcasted_iota(_i32, (E, E), 1)).astype(_f32)
    offs = jnp.dot(cnt_pad, up8, preferred_element_type=_f32)

    offs_b = jnp.broadcast_to(offs, (S, E))
    pos0_ref[...] = jnp.sum(m1 * (offs_b + c), axis=1, keepdims=True).astype(_i32)
    pos1_ref[...] = jnp.sum(m2 * (offs_b + c), axis=1, keepdims=True).astype(_i32)
    w1_ref[...] = s1
    w2_ref[...] = s2

    cnt_ref[...] = cnt.astype(_i32)
    p_e = jnp.sum(scores, axis=0, keepdims=True) / float(S)
    f_e = cnt / float(S * 2)
    lb_ref[...] = (E * jnp.sum(f_e * p_e, axis=1, keepdims=True) - 1.0)

    bid = (lax.broadcasted_iota(_i32, (64, E), 0) * BLK).astype(_f32)
    ind = (jnp.broadcast_to(offs, (64, E)) <= bid).astype(_i32)
    g_ref[...] = jnp.sum(ind, axis=1, keepdims=True) - 1


def _run_router(xf, router_W):
    outs = pl.pallas_call(
        _router_body,
        out_shape=(
            jax.ShapeDtypeStruct((S, 1), _i32),
            jax.ShapeDtypeStruct((S, 1), _i32),
            jax.ShapeDtypeStruct((S, 1), _f32),
            jax.ShapeDtypeStruct((S, 1), _f32),
            jax.ShapeDtypeStruct((1, E), _i32),
            jax.ShapeDtypeStruct((1, 1), _f32),
            jax.ShapeDtypeStruct((64, 1), _i32),
        ),
        scratch_shapes=[
            pltpu.VMEM((S, E), _f32),
            pltpu.VMEM((S, E), _f32),
        ],
    )(xf, router_W)
    return outs


_NTILES = 32
_TPC = S // 16
_SLOTS_PT = G // _NTILES
_GCH = 48
_GOFF = (0, 48, 96, 136)


def _clamp16(ref, n):
    for j in range(n // 16):
        sl = pl.ds(16 * j, 16)
        v = ref[sl]
        ref[sl] = jnp.minimum(jnp.maximum(v, 0), S - 1)


def _dispatch_body(x_hbm, p0_hbm, p1_hbm, w1_hbm, w2_hbm, gx_hbm, rw_hbm,
                   srcsh, rwsh, posv, valv, tokv, rwv,
                   ia, ib, ra, rb, sem):
    c = lax.axis_index("c")
    s = lax.axis_index("s")

    tb = s * _TPC
    for j in range(_TPC // 16):
        tokv[pl.ds(16 * j, 16)] = lax.iota(_i32, 16) + (tb + 16 * j)
    for (p_hbm, w_hbm) in ((p0_hbm, w1_hbm), (p1_hbm, w2_hbm)):
        pltpu.sync_copy(p_hbm.at[pl.ds(tb, _TPC)], posv)
        pltpu.sync_copy(w_hbm.at[pl.ds(tb, _TPC)], valv)
        pltpu.sync_copy(tokv, srcsh.at[posv])
        pltpu.sync_copy(valv, rwsh.at[posv])
    plsc.subcore_barrier()

    base = c * (G // 2) + s * _SLOTS_PT
    pltpu.sync_copy(rwsh.at[pl.ds(base, _SLOTS_PT)], rwv)
    pltpu.sync_copy(rwv, rw_hbm.at[pl.ds(base, _SLOTS_PT)])
    ibufs = (ia, ib)
    rbufs = (ra, rb)
    pltpu.sync_copy(srcsh.at[pl.ds(base + _GOFF[0], _GCH)], ia)
    _clamp16(ia, _GCH)
    cp = pltpu.async_copy(x_hbm.at[ia], ra, sem)
    for k in range(1, len(_GOFF)):
        nib = ibufs[k % 2]
        nrb = rbufs[k % 2]
        pltpu.sync_copy(srcsh.at[pl.ds(base + _GOFF[k], _GCH)], nib)
        _clamp16(nib, _GCH)
        cp.wait()
        cp = pltpu.async_copy(x_hbm.at[nib], nrb, sem)
        pltpu.sync_copy(rbufs[(k - 1) % 2],
                        gx_hbm.at[pl.ds(base + _GOFF[k - 1], _GCH)])
    cp.wait()
    pltpu.sync_copy(rbufs[(len(_GOFF) - 1) % 2],
                    gx_hbm.at[pl.ds(base + _GOFF[-1], _GCH)])


def _run_dispatch(xf, pos0, pos1, w1, w2):
    mesh = plsc.VectorSubcoreMesh(core_axis_name="c", subcore_axis_name="s")
    k = functools.partial(
        pl.kernel,
        mesh=mesh,
        out_type=(
            jax.ShapeDtypeStruct((G, D), _f32),
            jax.ShapeDtypeStruct((G,), _f32),
        ),
        scratch_types=[
            pltpu.VMEM_SHARED((G,), _i32),
            pltpu.VMEM_SHARED((G,), _f32),
            pltpu.VMEM((_TPC,), _i32),
            pltpu.VMEM((_TPC,), _f32),
            pltpu.VMEM((_TPC,), _i32),
            pltpu.VMEM((_SLOTS_PT,), _f32),
            pltpu.VMEM((_GCH,), _i32),
            pltpu.VMEM((_GCH,), _i32),
            pltpu.VMEM((_GCH, D), _f32),
            pltpu.VMEM((_GCH, D), _f32),
            pltpu.SemaphoreType.DMA,
        ],
    )(_dispatch_body)
    return k(xf, pos0, pos1, w1, w2)


_bf16 = jnp.bfloat16


def _k1_body(g_ref, gx_ref, wfc_ref, wg_ref, h_ref):
    xb = gx_ref[...].astype(_bf16)
    a = jnp.dot(xb, wfc_ref[0].astype(_bf16), preferred_element_type=_f32)
    z = jnp.dot(xb, wg_ref[0].astype(_bf16), preferred_element_type=_f32)
    h_ref[...] = (a * z * (1.0 / (1.0 + jnp.exp(-z)))).astype(_bf16)


def _run_k1(g, gx, Wfc, Wg):
    return pl.pallas_call(
        _k1_body,
        grid_spec=pltpu.PrefetchScalarGridSpec(
            num_scalar_prefetch=1,
            grid=(NH, NB),
            in_specs=[
                pl.BlockSpec((BLK, D), lambda h, b, g: (b, 0)),
                pl.BlockSpec((1, D, HT), lambda h, b, g: (g[b], 0, h)),
                pl.BlockSpec((1, D, HT), lambda h, b, g: (g[b], 0, h)),
            ],
            out_specs=pl.BlockSpec((BLK, HT), lambda h, b, g: (b, h)),
        ),
        out_shape=jax.ShapeDtypeStruct((G, H), _bf16),
    )(g, gx, Wfc, Wg)


def _k2_body(g_ref, h_ref, wp_ref, rw_ref, r_ref):
    r = jnp.dot(h_ref[...], wp_ref[0].astype(_bf16),
                preferred_element_type=_f32)
    r_ref[...] = r * rw_ref[:, 0:1]


def _run_k2(g, Hm, Wp, rw2d):
    return pl.pallas_call(
        _k2_body,
        grid_spec=pltpu.PrefetchScalarGridSpec(
            num_scalar_prefetch=1,
            grid=(NB,),
            in_specs=[
                pl.BlockSpec((BLK, H), lambda b, g: (b, 0)),
                pl.BlockSpec((1, H, D), lambda b, g: (g[b], 0, 0)),
                pl.BlockSpec((BLK, 128), lambda b, g: (b, 0)),
            ],
            out_specs=pl.BlockSpec((BLK, D), lambda b, g: (b, 0)),
        ),
        out_shape=jax.ShapeDtypeStruct((G, D), _f32),
    )(g, Hm, Wp, rw2d)


def _ks1_body(x_ref, wfc_ref, wg_ref, h_ref):
    xb = x_ref[...].astype(_bf16)
    a = jnp.dot(xb, wfc_ref[...].astype(_bf16), preferred_element_type=_f32)
    z = jnp.dot(xb, wg_ref[...].astype(_bf16), preferred_element_type=_f32)
    h_ref[...] = (a * z * (1.0 / (1.0 + jnp.exp(-z)))).astype(_bf16)


def _run_ks1(xf, Wsfc, Wsg):
    return pl.pallas_call(
        _ks1_body,
        grid=(NH, NSB),
        in_specs=[
            pl.BlockSpec((SBLK, D), lambda h, b: (b, 0)),
            pl.BlockSpec((D, HT), lambda h, b: (0, h)),
            pl.BlockSpec((D, HT), lambda h, b: (0, h)),
        ],
        out_specs=pl.BlockSpec((SBLK, HT), lambda h, b: (b, h)),
        out_shape=jax.ShapeDtypeStruct((S, H), _bf16),
    )(xf, Wsfc, Wsg)


def _ks2_body(h_ref, wp_ref, r_ref):
    r_ref[...] = jnp.dot(h_ref[...], wp_ref[...].astype(_bf16),
                         preferred_element_type=_f32)


def _run_ks2(Hs, Wsp):
    return pl.pallas_call(
        _ks2_body,
        grid=(NSB,),
        in_specs=[
            pl.BlockSpec((SBLK, H), lambda b: (b, 0)),
            pl.BlockSpec((H, D), lambda b: (0, 0)),
        ],
        out_specs=pl.BlockSpec((SBLK, D), lambda b: (b, 0)),
        out_shape=jax.ShapeDtypeStruct((S, D), _f32),
    )(Hs, Wsp)


_TOK_PT = S // _NTILES
_CCH = 16


def _combine_body(r_hbm, rs_hbm, p0_hbm, p1_hbm, y_hbm,
                  idx0, idx1, b0, b1, bs, sem):
    c = lax.axis_index("c")
    s = lax.axis_index("s")
    base = (s * 2 + c) * _TOK_PT
    for k in range(_TOK_PT // _CCH):
        tb = base + k * _CCH
        pltpu.sync_copy(p0_hbm.at[pl.ds(tb, _CCH)], idx0)
        pltpu.sync_copy(p1_hbm.at[pl.ds(tb, _CCH)], idx1)
        cp0 = pltpu.async_copy(r_hbm.at[idx0], b0, sem)
        cp1 = pltpu.async_copy(r_hbm.at[idx1], b1, sem)
        pltpu.sync_copy(rs_hbm.at[pl.ds(tb, _CCH)], bs)
        cp0.wait()
        cp1.wait()
        for i in range(_CCH):
            def _add(j, _, i=i):
                for u in range(4):
                    sl = pl.ds((j * 4 + u) * 16, 16)
                    b0[i, sl] = b0[i, sl] + b1[i, sl] + bs[i, sl]
                return 0
            lax.fori_loop(0, D // 64, _add, 0)
        pltpu.sync_copy(b0, y_hbm.at[pl.ds(tb, _CCH)])


def _run_combine(R, Rs, pos0, pos1):
    mesh = plsc.VectorSubcoreMesh(core_axis_name="c", subcore_axis_name="s")
    k = functools.partial(
        pl.kernel,
        mesh=mesh,
        out_type=jax.ShapeDtypeStruct((S, D), _f32),
        scratch_types=[
            pltpu.VMEM((_CCH,), _i32),
            pltpu.VMEM((_CCH,), _i32),
            pltpu.VMEM((_CCH, D), _f32),
            pltpu.VMEM((_CCH, D), _f32),
            pltpu.VMEM((_CCH, D), _f32),
            pltpu.SemaphoreType.DMA,
        ],
    )(_combine_body)
    return k(R, Rs, pos0, pos1)


def kernel(x, router_W, Wfc, Wg, Wp, Wsfc, Wsg, Wsp):
    B, S_, D_ = x.shape
    xf = x.reshape(S_, D_)

    (pos0c, pos1c, w1c, w2c, cnt2, lb2, g2) = _run_router(xf, router_W)
    pos0 = pos0c.reshape(S)
    pos1 = pos1c.reshape(S)
    g = g2.reshape(64)[:NB]

    gx, rw = _run_dispatch(xf, pos0, pos1, w1c.reshape(S), w2c.reshape(S))
    rw2d = jnp.broadcast_to(rw[:, None], (G, 128))

    Hs = _run_ks1(xf, Wsfc, Wsg)
    Rs = _run_ks2(Hs, Wsp)

    Hm = _run_k1(g, gx, Wfc, Wg)
    R = _run_k2(g, Hm, Wp, rw2d)

    y2 = _run_combine(R, Rs, pos0, pos1)

    y = y2.reshape(B, S_, D_)
    lb_loss = lb2.reshape(())
    counts = cnt2.reshape(E)
    return (y, lb_loss, counts)


# speedup vs baseline: 1.5950x; 1.0432x over previous
import functools

import jax
import jax.numpy as jnp
from jax import lax
from jax.experimental import pallas as pl
from jax.experimental.pallas import tpu as pltpu
from jax.experimental.pallas import tpu_sc as plsc

D = 1024
E = 8
H = 4096
S = 2048
BLK = 256
NB = 23
G = NB * BLK
HT = 2048
NH = H // HT
SBLK = 512
NSB = S // SBLK

_f32 = jnp.float32
_i32 = jnp.int32


def _router_body(x_ref, w_ref, posi_ref, wi_ref,
                 cnt_ref, lb_ref, g_ref, m_ref, c_ref):
    x = x_ref[...]
    logits = jnp.dot(x, w_ref[...], preferred_element_type=_f32)
    mx = jnp.max(logits, axis=1, keepdims=True)
    ex = jnp.exp(logits - mx)
    scores = ex / jnp.sum(ex, axis=1, keepdims=True)

    e_ids = lax.broadcasted_iota(_i32, (S, E), 1)
    s1 = jnp.max(scores, axis=1, keepdims=True)
    i1 = jnp.min(jnp.where(scores == s1, e_ids, E), axis=1, keepdims=True)
    masked = jnp.where(e_ids == i1, -1.0, scores)
    s2 = jnp.max(masked, axis=1, keepdims=True)
    i2 = jnp.min(jnp.where(masked == s2, e_ids, E), axis=1, keepdims=True)

    m1 = (e_ids == i1).astype(_f32)
    m2 = (e_ids == i2).astype(_f32)
    m_ref[...] = m1 + m2

    CH = 256
    for i in range(S // CH):
        rows = lax.broadcasted_iota(_i32, (CH, S), 0) + i * CH
        cols = lax.broadcasted_iota(_i32, (CH, S), 1)
        tril = (cols < rows).astype(_f32)
        c_ref[pl.ds(i * CH, CH), :] = jnp.dot(
            tril, m_ref[...], preferred_element_type=_f32)
    c = c_ref[...]

    cnt = jnp.sum(m_ref[...], axis=0, keepdims=True)
    cnt_pad = jnp.ceil(cnt / BLK) * BLK
    up8 = (lax.broadcasted_iota(_i32, (E, E), 0)
           < lax.broadcasted_iota(_i32, (E, E), 1)).astype(_f32)
    offs = jnp.dot(cnt_pad, up8, preferred_element_type=_f32)

    offs_b = jnp.broadcast_to(offs, (S, E))
    p0 = jnp.sum(m1 * (offs_b + c), axis=1, keepdims=True)
    p1 = jnp.sum(m2 * (offs_b + c), axis=1, keepdims=True)
    posi_ref[...] = jnp.concatenate([p0, p1], axis=1).astype(_i32)
    wi_ref[...] = jnp.concatenate([s1, s2], axis=1)

    cnt_ref[...] = cnt.astype(_i32)
    p_e = jnp.sum(scores, axis=0, keepdims=True) / float(S)
    f_e = cnt / float(S * 2)
    lb_ref[...] = (E * jnp.sum(f_e * p_e, axis=1, keepdims=True) - 1.0)

    bid = (lax.broadcasted_iota(_i32, (64, E), 0) * BLK).astype(_f32)
    ind = (jnp.broadcast_to(offs, (64, E)) <= bid).astype(_i32)
    g_ref[...] = jnp.sum(ind, axis=1, keepdims=True) - 1


def _run_router(xf, router_W):
    outs = pl.pallas_call(
        _router_body,
        out_shape=(
            jax.ShapeDtypeStruct((S, 2), _i32),
            jax.ShapeDtypeStruct((S, 2), _f32),
            jax.ShapeDtypeStruct((1, E), _i32),
            jax.ShapeDtypeStruct((1, 1), _f32),
            jax.ShapeDtypeStruct((64, 1), _i32),
        ),
        scratch_shapes=[
            pltpu.VMEM((S, E), _f32),
            pltpu.VMEM((S, E), _f32),
        ],
    )(xf, router_W)
    return outs


_NTILES = 32
_APC = 2 * S // 16
_SLOTS_PT = G // _NTILES
_GCH = 48
_GOFF = (0, 48, 96, 136)


def _clamp16(ref, n):
    for j in range(n // 16):
        sl = pl.ds(16 * j, 16)
        v = ref[sl]
        ref[sl] = jnp.minimum(jnp.maximum(v, 0), S - 1)


def _dispatch_body(x_hbm, pa_hbm, wa_hbm, gx_hbm, rw_hbm,
                   srcsh, rwsh, posv, valv, tokv, rwv,
                   ia, ib, ra, rb, sem):
    c = lax.axis_index("c")
    s = lax.axis_index("s")

    ab = s * _APC
    for j in range(_APC // 16):
        aid = lax.iota(_i32, 16) + (ab + 16 * j)
        tokv[pl.ds(16 * j, 16)] = lax.shift_right_logical(aid, 1)
    pltpu.sync_copy(pa_hbm.at[pl.ds(ab, _APC)], posv)
    pltpu.sync_copy(wa_hbm.at[pl.ds(ab, _APC)], valv)
    pltpu.sync_copy(tokv, srcsh.at[posv])
    pltpu.sync_copy(valv, rwsh.at[posv])
    plsc.subcore_barrier()

    base = c * (G // 2) + s * _SLOTS_PT
    pltpu.sync_copy(rwsh.at[pl.ds(base, _SLOTS_PT)], rwv)
    pltpu.sync_copy(rwv, rw_hbm.at[pl.ds(base, _SLOTS_PT)])
    ibufs = (ia, ib)
    rbufs = (ra, rb)
    pltpu.sync_copy(srcsh.at[pl.ds(base + _GOFF[0], _GCH)], ia)
    _clamp16(ia, _GCH)
    cp = pltpu.async_copy(x_hbm.at[ia], ra, sem)
    for k in range(1, len(_GOFF)):
        nib = ibufs[k % 2]
        nrb = rbufs[k % 2]
        pltpu.sync_copy(srcsh.at[pl.ds(base + _GOFF[k], _GCH)], nib)
        _clamp16(nib, _GCH)
        cp.wait()
        cp = pltpu.async_copy(x_hbm.at[nib], nrb, sem)
        pltpu.sync_copy(rbufs[(k - 1) % 2],
                        gx_hbm.at[pl.ds(base + _GOFF[k - 1], _GCH)])
    cp.wait()
    pltpu.sync_copy(rbufs[(len(_GOFF) - 1) % 2],
                    gx_hbm.at[pl.ds(base + _GOFF[-1], _GCH)])


def _run_dispatch(xf, pa, wa):
    mesh = plsc.VectorSubcoreMesh(core_axis_name="c", subcore_axis_name="s")
    k = functools.partial(
        pl.kernel,
        mesh=mesh,
        out_type=(
            jax.ShapeDtypeStruct((G, D), _f32),
            jax.ShapeDtypeStruct((G,), _f32),
        ),
        scratch_types=[
            pltpu.VMEM_SHARED((G,), _i32),
            pltpu.VMEM_SHARED((G,), _f32),
            pltpu.VMEM((_APC,), _i32),
            pltpu.VMEM((_APC,), _f32),
            pltpu.VMEM((_APC,), _i32),
            pltpu.VMEM((_SLOTS_PT,), _f32),
            pltpu.VMEM((_GCH,), _i32),
            pltpu.VMEM((_GCH,), _i32),
            pltpu.VMEM((_GCH, D), _f32),
            pltpu.VMEM((_GCH, D), _f32),
            pltpu.SemaphoreType.DMA,
        ],
    )(_dispatch_body)
    return k(xf, pa, wa)


_bf16 = jnp.bfloat16


def _k1_body(g_ref, gx_ref, wfc_ref, wg_ref, h_ref):
    xb = gx_ref[...].astype(_bf16)
    a = jnp.dot(xb, wfc_ref[0].astype(_bf16), preferred_element_type=_f32)
    z = jnp.dot(xb, wg_ref[0].astype(_bf16), preferred_element_type=_f32)
    h_ref[...] = (a * z * (1.0 / (1.0 + jnp.exp(-z)))).astype(_bf16)


def _run_k1(g, gx, Wfc, Wg):
    return pl.pallas_call(
        _k1_body,
        grid_spec=pltpu.PrefetchScalarGridSpec(
            num_scalar_prefetch=1,
            grid=(NH, NB),
            in_specs=[
                pl.BlockSpec((BLK, D), lambda h, b, g: (b, 0)),
                pl.BlockSpec((1, D, HT), lambda h, b, g: (g[b], 0, h)),
                pl.BlockSpec((1, D, HT), lambda h, b, g: (g[b], 0, h)),
            ],
            out_specs=pl.BlockSpec((BLK, HT), lambda h, b, g: (b, h)),
        ),
        out_shape=jax.ShapeDtypeStruct((G, H), _bf16),
    )(g, gx, Wfc, Wg)


def _k2_body(g_ref, h_ref, wp_ref, rw_ref, r_ref):
    r = jnp.dot(h_ref[...], wp_ref[0].astype(_bf16),
                preferred_element_type=_f32)
    r_ref[...] = r * rw_ref[:, 0:1]


def _run_k2(g, Hm, Wp, rw2d):
    return pl.pallas_call(
        _k2_body,
        grid_spec=pltpu.PrefetchScalarGridSpec(
            num_scalar_prefetch=1,
            grid=(NB,),
            in_specs=[
                pl.BlockSpec((BLK, H), lambda b, g: (b, 0)),
                pl.BlockSpec((1, H, D), lambda b, g: (g[b], 0, 0)),
                pl.BlockSpec((BLK, 128), lambda b, g: (b, 0)),
            ],
            out_specs=pl.BlockSpec((BLK, D), lambda b, g: (b, 0)),
        ),
        out_shape=jax.ShapeDtypeStruct((G, D), _f32),
    )(g, Hm, Wp, rw2d)


def _ks1_body(x_ref, wfc_ref, wg_ref, h_ref):
    xb = x_ref[...].astype(_bf16)
    a = jnp.dot(xb, wfc_ref[...].astype(_bf16), preferred_element_type=_f32)
    z = jnp.dot(xb, wg_ref[...].astype(_bf16), preferred_element_type=_f32)
    h_ref[...] = (a * z * (1.0 / (1.0 + jnp.exp(-z)))).astype(_bf16)


def _run_ks1(xf, Wsfc, Wsg):
    return pl.pallas_call(
        _ks1_body,
        grid=(NH, NSB),
        in_specs=[
            pl.BlockSpec((SBLK, D), lambda h, b: (b, 0)),
            pl.BlockSpec((D, HT), lambda h, b: (0, h)),
            pl.BlockSpec((D, HT), lambda h, b: (0, h)),
        ],
        out_specs=pl.BlockSpec((SBLK, HT), lambda h, b: (b, h)),
        out_shape=jax.ShapeDtypeStruct((S, H), _bf16),
    )(xf, Wsfc, Wsg)


def _ks2_body(h_ref, wp_ref, r_ref):
    r_ref[...] = jnp.dot(h_ref[...], wp_ref[...].astype(_bf16),
                         preferred_element_type=_f32)


def _run_ks2(Hs, Wsp):
    return pl.pallas_call(
        _ks2_body,
        grid=(NSB,),
        in_specs=[
            pl.BlockSpec((SBLK, H), lambda b: (b, 0)),
            pl.BlockSpec((H, D), lambda b: (0, 0)),
        ],
        out_specs=pl.BlockSpec((SBLK, D), lambda b: (b, 0)),
        out_shape=jax.ShapeDtypeStruct((S, D), _f32),
    )(Hs, Wsp)


_TOK_PT = S // _NTILES
_CCH = 16


def _combine_body(r_hbm, rs_hbm, p0_hbm, p1_hbm, y_hbm,
                  idx0, idx1, b0, b1, bs, sem):
    c = lax.axis_index("c")
    s = lax.axis_index("s")
    base = (s * 2 + c) * _TOK_PT
    for k in range(_TOK_PT // _CCH):
        tb = base + k * _CCH
        pltpu.sync_copy(p0_hbm.at[pl.ds(tb, _CCH)], idx0)
        pltpu.sync_copy(p1_hbm.at[pl.ds(tb, _CCH)], idx1)
        cp0 = pltpu.async_copy(r_hbm.at[idx0], b0, sem)
        cp1 = pltpu.async_copy(r_hbm.at[idx1], b1, sem)
        pltpu.sync_copy(rs_hbm.at[pl.ds(tb, _CCH)], bs)
        cp0.wait()
        cp1.wait()
        for i in range(_CCH):
            def _add(j, _, i=i):
                for u in range(4):
                    sl = pl.ds((j * 4 + u) * 16, 16)
                    b0[i, sl] = b0[i, sl] + b1[i, sl] + bs[i, sl]
                return 0
            lax.fori_loop(0, D // 64, _add, 0)
        pltpu.sync_copy(b0, y_hbm.at[pl.ds(tb, _CCH)])


def _run_combine(R, Rs, pos0, pos1):
    mesh = plsc.VectorSubcoreMesh(core_axis_name="c", subcore_axis_name="s")
    k = functools.partial(
        pl.kernel,
        mesh=mesh,
        out_type=jax.ShapeDtypeStruct((S, D), _f32),
        scratch_types=[
            pltpu.VMEM((_CCH,), _i32),
            pltpu.VMEM((_CCH,), _i32),
            pltpu.VMEM((_CCH, D), _f32),
            pltpu.VMEM((_CCH, D), _f32),
            pltpu.VMEM((_CCH, D), _f32),
            pltpu.SemaphoreType.DMA,
        ],
    )(_combine_body)
    return k(R, Rs, pos0, pos1)


def kernel(x, router_W, Wfc, Wg, Wp, Wsfc, Wsg, Wsp):
    B, S_, D_ = x.shape
    xf = x.reshape(S_, D_)

    (posi, wi, cnt2, lb2, g2) = _run_router(xf, router_W)
    pos0 = posi[:, 0]
    pos1 = posi[:, 1]
    g = g2.reshape(64)[:NB]

    gx, rw = _run_dispatch(xf, posi.reshape(2 * S), wi.reshape(2 * S))
    rw2d = jnp.broadcast_to(rw[:, None], (G, 128))

    Hs = _run_ks1(xf, Wsfc, Wsg)
    Rs = _run_ks2(Hs, Wsp)

    Hm = _run_k1(g, gx, Wfc, Wg)
    R = _run_k2(g, Hm, Wp, rw2d)

    y2 = _run_combine(R, Rs, pos0, pos1)

    y = y2.reshape(B, S_, D_)
    lb_loss = lb2.reshape(())
    counts = cnt2.reshape(E)
    return (y, lb_loss, counts)
